# baseline (speedup 1.0000x reference)
import functools

import jax
import jax.numpy as jnp
from jax.experimental import pallas as pl

N = 10000
E = 320000
D_IN = 128
HID = 128
CLS = 64
K = 20
GAMMA = 0.1


def _matmul_kernel(a_ref, b_ref, o_ref):
    o_ref[...] = jnp.dot(a_ref[...], b_ref[...],
                         preferred_element_type=jnp.float32)


def _pallas_matmul(a, b):
    m, k = a.shape
    k2, n = b.shape
    return pl.pallas_call(
        _matmul_kernel,
        out_shape=jax.ShapeDtypeStruct((m, n), jnp.float32),
        grid=(m // 400,),
        in_specs=[pl.BlockSpec((400, k), lambda i: (i, 0)),
                  pl.BlockSpec((k2, n), lambda i: (0, 0))],
        out_specs=pl.BlockSpec((400, n), lambda i: (i, 0)),
    )(a, b)


def _norm_adj(edge_index):
    src = edge_index[0]
    dst = edge_index[1]
    mask = (src != dst).astype(jnp.float32)
    loops = jnp.arange(N, dtype=src.dtype)
    src = jnp.concatenate([src, loops])
    dst = jnp.concatenate([dst, loops])
    w = jnp.concatenate([mask, jnp.ones((N,), jnp.float32)])
    deg = jnp.zeros((N,), jnp.float32).at[dst].add(w)
    deg = jnp.maximum(deg, 1.0)
    d_isqrt = deg ** -0.5
    vals = d_isqrt[src] * d_isqrt[dst] * w
    return src, dst, vals


def _spmm(src, dst, vals, H):
    return jnp.zeros((N, H.shape[1]), H.dtype).at[src].add(vals[:, None] * H[dst])


def _knn(x):
    Xn = x / jnp.maximum(jnp.linalg.norm(x, axis=1, keepdims=True), 1e-12)
    S = Xn @ Xn.T
    di = jnp.arange(N)
    Sk = S.at[di, di].set(-1.0)
    vals, idx = jax.lax.top_k(Sk, K)
    rows = jnp.repeat(jnp.arange(N), K)
    cols = idx.reshape(-1)
    sim = jnp.maximum(vals.reshape(-1), 0.0)
    r = jnp.concatenate([rows, cols])
    c = jnp.concatenate([cols, rows])
    v = jnp.concatenate([sim, sim])
    Drow = jnp.zeros((N,), jnp.float32).at[r].add(v)
    Dis = jnp.maximum(Drow, 1e-12) ** -0.5
    nv = Dis[r] * v * Dis[c]
    return r, c, nv


def kernel(features, edge_index, W0, b0, W1, b1, ws0, bs0, ws1, bs1, wk0, bk0, wk1, bk1):
    asrc, adst, avals = _norm_adj(edge_index)
    r, c, nv = _knn(features)
    H = features
    s = jax.nn.sigmoid(H @ ws0 + bs0)
    Kv = H @ wk0 + bk0
    AH = _spmm(asrc, adst, avals, H)
    SfH = _spmm(r, c, nv, H)
    PH = s[:, None] * AH + (1.0 - s)[:, None] * SfH + GAMMA * (Kv[:, None] * H)
    H = _pallas_matmul(PH, W0) + b0
    H = jax.nn.relu(H)
    s = jax.nn.sigmoid(H @ ws1 + bs1)
    Kv = H @ wk1 + bk1
    AH = _spmm(asrc, adst, avals, H)
    SfH = _spmm(r, c, nv, H)
    PH = s[:, None] * AH + (1.0 - s)[:, None] * SfH + GAMMA * (Kv[:, None] * H)
    H = _pallas_matmul(PH, W1) + b1
    return H


# speedup vs baseline: 1.5993x; 1.5993x over previous
import functools

import jax
import jax.numpy as jnp
from jax.experimental import pallas as pl
from jax.experimental.pallas import tpu as pltpu

N = 10000
E = 320000
D_IN = 128
HID = 128
CLS = 64
K = 20
GAMMA = 0.1

NPAD = 10240
RB = 128


def _normalize_kernel(x_ref, o_ref):
    x = x_ref[...]
    s = jnp.sum(x * x, axis=1, keepdims=True)
    nrm = jnp.maximum(jnp.sqrt(s), 1e-12)
    o_ref[...] = x / nrm


def _row_normalize(x):
    n, d = x.shape
    blk = 400
    return pl.pallas_call(
        _normalize_kernel,
        out_shape=jax.ShapeDtypeStruct((n, d), jnp.float32),
        grid=(n // blk,),
        in_specs=[pl.BlockSpec((blk, d), lambda i: (i, 0))],
        out_specs=pl.BlockSpec((blk, d), lambda i: (i, 0)),
    )(x)


def _topk_kernel(xr_ref, xt_ref, topv_ref, topi_ref, s_ref, *, npad, k):
    i = pl.program_id(0)
    rows = xr_ref[...]
    xt = xt_ref[...]
    s = jnp.dot(rows, xt, preferred_element_type=jnp.float32)
    col = jax.lax.broadcasted_iota(jnp.int32, (RB, npad), 1)
    grow = i * RB + jax.lax.broadcasted_iota(jnp.int32, (RB, npad), 0)
    s = jnp.where(col == grow, -1.0, s)
    s_ref[...] = s
    vals = []
    idxs = []
    big = jnp.int32(2**30)
    for _ in range(k):
        s = s_ref[...]
        gm = jnp.max(s, axis=1, keepdims=True)
        eq = s == gm
        cand = jnp.where(eq, col, big)
        aidx = jnp.min(cand, axis=1, keepdims=True)
        s_ref[...] = jnp.where(col == aidx, -2.0, s)
        vals.append(gm)
        idxs.append(aidx)
    pad_v = jnp.zeros((RB, 128 - k), jnp.float32)
    pad_i = jnp.zeros((RB, 128 - k), jnp.int32)
    topv_ref[...] = jnp.concatenate(vals + [pad_v], axis=1)
    topi_ref[...] = jnp.concatenate(idxs + [pad_i], axis=1)


def _topk_sim(xn_pad):
    xt = xn_pad.T
    kern = functools.partial(_topk_kernel, npad=NPAD, k=K)
    return pl.pallas_call(
        kern,
        out_shape=(jax.ShapeDtypeStruct((NPAD, 128), jnp.float32),
                   jax.ShapeDtypeStruct((NPAD, 128), jnp.int32)),
        grid=(NPAD // RB,),
        in_specs=[pl.BlockSpec((RB, D_IN), lambda i: (i, 0)),
                  pl.BlockSpec((D_IN, NPAD), lambda i: (0, 0))],
        out_specs=(pl.BlockSpec((RB, 128), lambda i: (i, 0)),
                   pl.BlockSpec((RB, 128), lambda i: (i, 0))),
        scratch_shapes=[pltpu.VMEM((RB, NPAD), jnp.float32)],
    )(xn_pad, xt)


def _matmul_kernel(a_ref, b_ref, o_ref):
    o_ref[...] = jnp.dot(a_ref[...], b_ref[...],
                         preferred_element_type=jnp.float32)


def _pallas_matmul(a, b):
    m, k = a.shape
    k2, n = b.shape
    return pl.pallas_call(
        _matmul_kernel,
        out_shape=jax.ShapeDtypeStruct((m, n), jnp.float32),
        grid=(m // 400,),
        in_specs=[pl.BlockSpec((400, k), lambda i: (i, 0)),
                  pl.BlockSpec((k2, n), lambda i: (0, 0))],
        out_specs=pl.BlockSpec((400, n), lambda i: (i, 0)),
    )(a, b)


def _norm_adj(edge_index):
    src = edge_index[0]
    dst = edge_index[1]
    mask = (src != dst).astype(jnp.float32)
    loops = jnp.arange(N, dtype=src.dtype)
    src = jnp.concatenate([src, loops])
    dst = jnp.concatenate([dst, loops])
    w = jnp.concatenate([mask, jnp.ones((N,), jnp.float32)])
    deg = jnp.zeros((N,), jnp.float32).at[dst].add(w)
    deg = jnp.maximum(deg, 1.0)
    d_isqrt = deg ** -0.5
    vals = d_isqrt[src] * d_isqrt[dst] * w
    return src, dst, vals


def _spmm(src, dst, vals, H):
    return jnp.zeros((N, H.shape[1]), H.dtype).at[src].add(vals[:, None] * H[dst])


def _knn_graph(features):
    xn = _row_normalize(features)
    xn_pad = jnp.pad(xn, ((0, NPAD - N), (0, 0)))
    topv_p, topi_p = _topk_sim(xn_pad)
    topv = topv_p[:N, :K]
    topi = jnp.minimum(topi_p[:N, :K], N - 1)
    rows = jnp.repeat(jnp.arange(N), K)
    cols = topi.reshape(-1)
    sim = jnp.maximum(topv.reshape(-1), 0.0)
    r = jnp.concatenate([rows, cols])
    c = jnp.concatenate([cols, rows])
    v = jnp.concatenate([sim, sim])
    Drow = jnp.zeros((N,), jnp.float32).at[r].add(v)
    Dis = jnp.maximum(Drow, 1e-12) ** -0.5
    nv = Dis[r] * v * Dis[c]
    return r, c, nv


def kernel(features, edge_index, W0, b0, W1, b1, ws0, bs0, ws1, bs1, wk0, bk0, wk1, bk1):
    asrc, adst, avals = _norm_adj(edge_index)
    r, c, nv = _knn_graph(features)
    H = features
    s = jax.nn.sigmoid(H @ ws0 + bs0)
    Kv = H @ wk0 + bk0
    AH = _spmm(asrc, adst, avals, H)
    SfH = _spmm(r, c, nv, H)
    PH = s[:, None] * AH + (1.0 - s)[:, None] * SfH + GAMMA * (Kv[:, None] * H)
    H = _pallas_matmul(PH, W0) + b0
    H = jax.nn.relu(H)
    s = jax.nn.sigmoid(H @ ws1 + bs1)
    Kv = H @ wk1 + bk1
    AH = _spmm(asrc, adst, avals, H)
    SfH = _spmm(r, c, nv, H)
    PH = s[:, None] * AH + (1.0 - s)[:, None] * SfH + GAMMA * (Kv[:, None] * H)
    H = _pallas_matmul(PH, W1) + b1
    return H


# speedup vs baseline: 5.9299x; 3.7079x over previous
import functools

import jax
import jax.numpy as jnp
from jax import lax
from jax.experimental import pallas as pl
from jax.experimental.pallas import tpu as pltpu
from jax.experimental.pallas import tpu_sc as plsc

N = 10000
E = 320000
D_IN = 128
HID = 128
CLS = 64
K = 20
GAMMA = 0.1

NPAD = 10240
RB = 128


def _normalize_kernel(x_ref, o_ref):
    x = x_ref[...]
    s = jnp.sum(x * x, axis=1, keepdims=True)
    nrm = jnp.maximum(jnp.sqrt(s), 1e-12)
    o_ref[...] = x / nrm


def _row_normalize(x):
    n, d = x.shape
    blk = 400
    return pl.pallas_call(
        _normalize_kernel,
        out_shape=jax.ShapeDtypeStruct((n, d), jnp.float32),
        grid=(n // blk,),
        in_specs=[pl.BlockSpec((blk, d), lambda i: (i, 0))],
        out_specs=pl.BlockSpec((blk, d), lambda i: (i, 0)),
    )(x)


def _topk_kernel(xr_ref, xt_ref, topv_ref, topi_ref, s_ref, *, npad, k):
    i = pl.program_id(0)
    rows = xr_ref[...]
    xt = xt_ref[...]
    s = jnp.dot(rows, xt, preferred_element_type=jnp.float32)
    col = jax.lax.broadcasted_iota(jnp.int32, (RB, npad), 1)
    grow = i * RB + jax.lax.broadcasted_iota(jnp.int32, (RB, npad), 0)
    s = jnp.where(col == grow, -1.0, s)
    s_ref[...] = s
    vals = []
    idxs = []
    big = jnp.int32(2**30)
    for _ in range(k):
        s = s_ref[...]
        gm = jnp.max(s, axis=1, keepdims=True)
        eq = s == gm
        cand = jnp.where(eq, col, big)
        aidx = jnp.min(cand, axis=1, keepdims=True)
        s_ref[...] = jnp.where(col == aidx, -2.0, s)
        vals.append(gm)
        idxs.append(aidx)
    pad_v = jnp.zeros((RB, 128 - k), jnp.float32)
    pad_i = jnp.zeros((RB, 128 - k), jnp.int32)
    topv_ref[...] = jnp.concatenate(vals + [pad_v], axis=1)
    topi_ref[...] = jnp.concatenate(idxs + [pad_i], axis=1)


def _topk_sim(xn_pad):
    xt = xn_pad.T
    kern = functools.partial(_topk_kernel, npad=NPAD, k=K)
    return pl.pallas_call(
        kern,
        out_shape=(jax.ShapeDtypeStruct((NPAD, 128), jnp.float32),
                   jax.ShapeDtypeStruct((NPAD, 128), jnp.int32)),
        grid=(NPAD // RB,),
        in_specs=[pl.BlockSpec((RB, D_IN), lambda i: (i, 0)),
                  pl.BlockSpec((D_IN, NPAD), lambda i: (0, 0))],
        out_specs=(pl.BlockSpec((RB, 128), lambda i: (i, 0)),
                   pl.BlockSpec((RB, 128), lambda i: (i, 0))),
        scratch_shapes=[pltpu.VMEM((RB, NPAD), jnp.float32)],
    )(xn_pad, xt)


def _matmul_kernel(a_ref, b_ref, o_ref):
    o_ref[...] = jnp.dot(a_ref[...], b_ref[...],
                         preferred_element_type=jnp.float32)


def _pallas_matmul(a, b):
    m, k = a.shape
    k2, n = b.shape
    return pl.pallas_call(
        _matmul_kernel,
        out_shape=jax.ShapeDtypeStruct((m, n), jnp.float32),
        grid=(m // 400,),
        in_specs=[pl.BlockSpec((400, k), lambda i: (i, 0)),
                  pl.BlockSpec((k2, n), lambda i: (0, 0))],
        out_specs=pl.BlockSpec((400, n), lambda i: (i, 0)),
    )(a, b)


ACC_ROWS = 10240
ROWS_PER_SUB = ACC_ROWS // 16
CHUNK = 128
EA_PAD = 321536
ES_PAD = 401408
NCH_A = EA_PAD // (16 * CHUNK)
NCH_S = ES_PAD // (16 * CHUNK)

_sc_mesh = plsc.VectorSubcoreMesh(core_axis_name="c", subcore_axis_name="s")


def _spmm_sc_body(gA_hbm, gS_hbm, arow_hbm, acol_hbm, srow_hbm, scol_hbm,
                  sval_hbm, zeros_hbm, outA_hbm, outS_hbm,
                  idxr_v, idxc_v, val_v, rows_v, acc_sh, sem):
    c = lax.axis_index("c")
    s = lax.axis_index("s")
    base_row = s * ROWS_PER_SUB
    pltpu.sync_copy(zeros_hbm, acc_sh.at[pl.ds(base_row, ROWS_PER_SUB)])
    plsc.subcore_barrier()

    def run_edges(row_hbm, col_hbm, n_chunks, scaled):
        def chunk_body(g, _):
            base = (s * n_chunks + g) * CHUNK
            pltpu.sync_copy(row_hbm.at[pl.ds(base, CHUNK)], idxr_v)
            pltpu.sync_copy(col_hbm.at[pl.ds(base, CHUNK)], idxc_v)
            pltpu.async_copy(gA_hbm.at[idxc_v] if not scaled
                             else gS_hbm.at[idxc_v], rows_v, sem).wait()
            if scaled:
                pltpu.sync_copy(sval_hbm.at[pl.ds(base, CHUNK)], val_v)
                for t in range(CHUNK // 16):
                    vv = val_v[pl.ds(t * 16, 16)]
                    for i in range(16):
                        v = vv[i]
                        e = t * 16 + i
                        for j in range(8):
                            sl = pl.ds(j * 16, 16)
                            rows_v[e, sl] = rows_v[e, sl] * v
            pltpu.sync_copy(rows_v, acc_sh.at[idxr_v], add=True)
            return 0

        lax.fori_loop(0, n_chunks, chunk_body, 0)

    @pl.when(c == 0)
    def _():
        run_edges(arow_hbm, acol_hbm, NCH_A, scaled=False)

    @pl.when(c == 1)
    def _():
        run_edges(srow_hbm, scol_hbm, NCH_S, scaled=True)

    plsc.subcore_barrier()
    src_slab = acc_sh.at[pl.ds(base_row, ROWS_PER_SUB)]

    @pl.when(c == 0)
    def _():
        pltpu.sync_copy(src_slab, outA_hbm.at[pl.ds(base_row, ROWS_PER_SUB)])

    @pl.when(c == 1)
    def _():
        pltpu.sync_copy(src_slab, outS_hbm.at[pl.ds(base_row, ROWS_PER_SUB)])


_spmm_sc = pl.kernel(
    _spmm_sc_body,
    out_type=(jax.ShapeDtypeStruct((ACC_ROWS, 128), jnp.float32),
              jax.ShapeDtypeStruct((ACC_ROWS, 128), jnp.float32)),
    mesh=_sc_mesh,
    scratch_types=[
        pltpu.VMEM((CHUNK,), jnp.int32),
        pltpu.VMEM((CHUNK,), jnp.int32),
        pltpu.VMEM((CHUNK,), jnp.float32),
        pltpu.VMEM((CHUNK, 128), jnp.float32),
        pltpu.VMEM_SHARED((ACC_ROWS, 128), jnp.float32),
        pltpu.SemaphoreType.DMA,
    ],
)


def _norm_adj_edges(edge_index):
    src = edge_index[0].astype(jnp.int32)
    dst = edge_index[1].astype(jnp.int32)
    self_loop = src == dst
    arow = jnp.where(self_loop, N, src)
    acol = dst
    arow = jnp.concatenate([arow, jnp.full((EA_PAD - E,), N, jnp.int32)])
    acol = jnp.concatenate([acol, jnp.zeros((EA_PAD - E,), jnp.int32)])
    deg = jnp.zeros((N,), jnp.float32).at[dst].add(
        (~self_loop).astype(jnp.float32)) + 1.0
    deg = jnp.maximum(deg, 1.0)
    d_isqrt = deg ** -0.5
    return arow, acol, d_isqrt


def _knn_graph(features):
    xn = _row_normalize(features)
    xn_pad = jnp.pad(xn, ((0, NPAD - N), (0, 0)))
    topv_p, topi_p = _topk_sim(xn_pad)
    topv = topv_p[:N, :K]
    topi = jnp.minimum(topi_p[:N, :K], N - 1)
    rows = jnp.repeat(jnp.arange(N, dtype=jnp.int32), K)
    cols = topi.reshape(-1)
    sim = jnp.maximum(topv.reshape(-1), 0.0)
    r = jnp.concatenate([rows, cols])
    c = jnp.concatenate([cols, rows])
    v = jnp.concatenate([sim, sim])
    Drow = jnp.zeros((N,), jnp.float32).at[r].add(v)
    Dis = jnp.maximum(Drow, 1e-12) ** -0.5
    srow = jnp.concatenate([r, jnp.full((ES_PAD - 2 * N * K,), N, jnp.int32)])
    scol = jnp.concatenate([c, jnp.zeros((ES_PAD - 2 * N * K,), jnp.int32)])
    sval = jnp.concatenate([v, jnp.zeros((ES_PAD - 2 * N * K,), jnp.float32)])
    return srow, scol, sval, Dis


_ZEROS_SLAB = None


def _propagate(H, arow, acol, d_isqrt, srow, scol, sval, Dis, zeros_slab):
    gA = d_isqrt[:, None] * H
    gS = Dis[:, None] * H
    outA, outS = _spmm_sc(gA, gS, arow, acol, srow, scol, sval, zeros_slab)
    AH = d_isqrt[:, None] * (outA[:N] + gA)
    SfH = Dis[:, None] * outS[:N]
    return AH, SfH


def kernel(features, edge_index, W0, b0, W1, b1, ws0, bs0, ws1, bs1, wk0, bk0, wk1, bk1):
    arow, acol, d_isqrt = _norm_adj_edges(edge_index)
    srow, scol, sval, Dis = _knn_graph(features)
    zeros_slab = jnp.zeros((ROWS_PER_SUB, 128), jnp.float32)
    H = features
    s = jax.nn.sigmoid(H @ ws0 + bs0)
    Kv = H @ wk0 + bk0
    AH, SfH = _propagate(H, arow, acol, d_isqrt, srow, scol, sval, Dis,
                         zeros_slab)
    PH = s[:, None] * AH + (1.0 - s)[:, None] * SfH + GAMMA * (Kv[:, None] * H)
    H = _pallas_matmul(PH, W0) + b0
    H = jax.nn.relu(H)
    s = jax.nn.sigmoid(H @ ws1 + bs1)
    Kv = H @ wk1 + bk1
    AH, SfH = _propagate(H, arow, acol, d_isqrt, srow, scol, sval, Dis,
                         zeros_slab)
    PH = s[:, None] * AH + (1.0 - s)[:, None] * SfH + GAMMA * (Kv[:, None] * H)
    H = _pallas_matmul(PH, W1) + b1
    return H


# speedup vs baseline: 6.6122x; 1.1151x over previous
import functools

import jax
import jax.numpy as jnp
from jax import lax
from jax.experimental import pallas as pl
from jax.experimental.pallas import tpu as pltpu
from jax.experimental.pallas import tpu_sc as plsc

N = 10000
E = 320000
D_IN = 128
HID = 128
CLS = 64
K = 20
GAMMA = 0.1

NPAD = 10240
RB = 128


def _normalize_kernel(x_ref, o_ref):
    x = x_ref[...]
    s = jnp.sum(x * x, axis=1, keepdims=True)
    nrm = jnp.maximum(jnp.sqrt(s), 1e-12)
    o_ref[...] = x / nrm


def _row_normalize(x):
    n, d = x.shape
    blk = 400
    return pl.pallas_call(
        _normalize_kernel,
        out_shape=jax.ShapeDtypeStruct((n, d), jnp.float32),
        grid=(n // blk,),
        in_specs=[pl.BlockSpec((blk, d), lambda i: (i, 0))],
        out_specs=pl.BlockSpec((blk, d), lambda i: (i, 0)),
    )(x)


def _topk_kernel(xr_ref, xt_ref, topv_ref, topi_ref, s_ref, *, npad, k):
    i = pl.program_id(0)
    rows = xr_ref[...]
    xt = xt_ref[...]
    s = jnp.dot(rows, xt, preferred_element_type=jnp.float32)
    col = jax.lax.broadcasted_iota(jnp.int32, (RB, npad), 1)
    grow = i * RB + jax.lax.broadcasted_iota(jnp.int32, (RB, npad), 0)
    s = jnp.where(col == grow, -1.0, s)
    s_ref[...] = s
    vals = []
    idxs = []
    big = jnp.int32(2**30)
    for _ in range(k):
        s = s_ref[...]
        gm = jnp.max(s, axis=1, keepdims=True)
        eq = s == gm
        cand = jnp.where(eq, col, big)
        aidx = jnp.min(cand, axis=1, keepdims=True)
        s_ref[...] = jnp.where(col == aidx, -2.0, s)
        vals.append(gm)
        idxs.append(aidx)
    pad_v = jnp.zeros((RB, 128 - k), jnp.float32)
    pad_i = jnp.zeros((RB, 128 - k), jnp.int32)
    topv_ref[...] = jnp.concatenate(vals + [pad_v], axis=1)
    topi_ref[...] = jnp.concatenate(idxs + [pad_i], axis=1)


def _topk_sim(xn_pad):
    xt = xn_pad.T
    kern = functools.partial(_topk_kernel, npad=NPAD, k=K)
    return pl.pallas_call(
        kern,
        out_shape=(jax.ShapeDtypeStruct((NPAD, 128), jnp.float32),
                   jax.ShapeDtypeStruct((NPAD, 128), jnp.int32)),
        grid=(NPAD // RB,),
        in_specs=[pl.BlockSpec((RB, D_IN), lambda i: (i, 0)),
                  pl.BlockSpec((D_IN, NPAD), lambda i: (0, 0))],
        out_specs=(pl.BlockSpec((RB, 128), lambda i: (i, 0)),
                   pl.BlockSpec((RB, 128), lambda i: (i, 0))),
        scratch_shapes=[pltpu.VMEM((RB, NPAD), jnp.float32)],
    )(xn_pad, xt)


def _matmul_kernel(a_ref, b_ref, o_ref):
    o_ref[...] = jnp.dot(a_ref[...], b_ref[...],
                         preferred_element_type=jnp.float32)


def _pallas_matmul(a, b):
    m, k = a.shape
    k2, n = b.shape
    return pl.pallas_call(
        _matmul_kernel,
        out_shape=jax.ShapeDtypeStruct((m, n), jnp.float32),
        grid=(m // 400,),
        in_specs=[pl.BlockSpec((400, k), lambda i: (i, 0)),
                  pl.BlockSpec((k2, n), lambda i: (0, 0))],
        out_specs=pl.BlockSpec((400, n), lambda i: (i, 0)),
    )(a, b)


ACC_ROWS = 10240
ROWS_PER_SUB = ACC_ROWS // 16
CHUNK = 128
EA_PAD = 321536
ES_PAD = 401408
NCH_A = EA_PAD // (16 * CHUNK)
NCH_S = ES_PAD // (16 * CHUNK)

_sc_mesh = plsc.VectorSubcoreMesh(core_axis_name="c", subcore_axis_name="s")


def _spmm_sc_body(gA_hbm, gS_hbm, arow_hbm, acol_hbm, srow_hbm, scol_hbm,
                  sval_hbm, zeros_hbm, outA_hbm, outS_hbm,
                  idxr_v, idxc_v, val_v, rows_v, acc_sh, sem):
    c = lax.axis_index("c")
    s = lax.axis_index("s")
    base_row = s * ROWS_PER_SUB
    pltpu.sync_copy(zeros_hbm, acc_sh.at[pl.ds(base_row, ROWS_PER_SUB)])
    plsc.subcore_barrier()

    def run_edges(row_hbm, col_hbm, n_chunks, scaled):
        def chunk_body(g, _):
            base = (s * n_chunks + g) * CHUNK
            pltpu.sync_copy(row_hbm.at[pl.ds(base, CHUNK)], idxr_v)
            pltpu.sync_copy(col_hbm.at[pl.ds(base, CHUNK)], idxc_v)
            pltpu.async_copy(gA_hbm.at[idxc_v] if not scaled
                             else gS_hbm.at[idxc_v], rows_v, sem).wait()
            if scaled:
                pltpu.sync_copy(sval_hbm.at[pl.ds(base, CHUNK)], val_v)
                for t in range(CHUNK // 16):
                    vv = val_v[pl.ds(t * 16, 16)]
                    for i in range(16):
                        v = vv[i]
                        e = t * 16 + i
                        for j in range(8):
                            sl = pl.ds(j * 16, 16)
                            rows_v[e, sl] = rows_v[e, sl] * v
            pltpu.sync_copy(rows_v, acc_sh.at[idxr_v], add=True)
            return 0

        lax.fori_loop(0, n_chunks, chunk_body, 0)

    @pl.when(c == 0)
    def _():
        run_edges(arow_hbm, acol_hbm, NCH_A, scaled=False)

    @pl.when(c == 1)
    def _():
        run_edges(srow_hbm, scol_hbm, NCH_S, scaled=True)

    plsc.subcore_barrier()
    src_slab = acc_sh.at[pl.ds(base_row, ROWS_PER_SUB)]

    @pl.when(c == 0)
    def _():
        pltpu.sync_copy(src_slab, outA_hbm.at[pl.ds(base_row, ROWS_PER_SUB)])

    @pl.when(c == 1)
    def _():
        pltpu.sync_copy(src_slab, outS_hbm.at[pl.ds(base_row, ROWS_PER_SUB)])


_spmm_sc = pl.kernel(
    _spmm_sc_body,
    out_type=(jax.ShapeDtypeStruct((ACC_ROWS, 128), jnp.float32),
              jax.ShapeDtypeStruct((ACC_ROWS, 128), jnp.float32)),
    mesh=_sc_mesh,
    scratch_types=[
        pltpu.VMEM((CHUNK,), jnp.int32),
        pltpu.VMEM((CHUNK,), jnp.int32),
        pltpu.VMEM((CHUNK,), jnp.float32),
        pltpu.VMEM((CHUNK, 128), jnp.float32),
        pltpu.VMEM_SHARED((ACC_ROWS, 128), jnp.float32),
        pltpu.SemaphoreType.DMA,
    ],
)


def _hist_sc_body(idx_hbm, val_hbm, zeros_hbm, out_hbm,
                  idx_v, val_v, rows_v, acc_sh, sem, *, n_chunks):
    c = lax.axis_index("c")
    s = lax.axis_index("s")
    base_row = s * ROWS_PER_SUB
    pltpu.sync_copy(zeros_hbm, acc_sh.at[pl.ds(base_row, ROWS_PER_SUB)])
    pltpu.sync_copy(zeros_hbm.at[pl.ds(0, CHUNK)], rows_v)
    plsc.subcore_barrier()
    w = s * 2 + c
    onehot = jnp.where(lax.iota(jnp.int32, 16) == 0, 1.0, 0.0)

    def chunk_body(g, _):
        base = (w * n_chunks + g) * CHUNK
        pltpu.sync_copy(idx_hbm.at[pl.ds(base, CHUNK)], idx_v)
        pltpu.sync_copy(val_hbm.at[pl.ds(base, CHUNK)], val_v)
        for t in range(CHUNK // 16):
            vv = val_v[pl.ds(t * 16, 16)]
            for i in range(16):
                rows_v[t * 16 + i, pl.ds(0, 16)] = vv[i] * onehot
        pltpu.sync_copy(rows_v, acc_sh.at[idx_v], add=True)
        return 0

    lax.fori_loop(0, n_chunks, chunk_body, 0)
    plsc.subcore_barrier()
    pltpu.sync_copy(acc_sh.at[pl.ds(base_row, ROWS_PER_SUB)],
                    out_hbm.at[c, pl.ds(base_row, ROWS_PER_SUB)])


def _make_hist_sc(n_chunks):
    return pl.kernel(
        functools.partial(_hist_sc_body, n_chunks=n_chunks),
        out_type=jax.ShapeDtypeStruct((2, ACC_ROWS, 128), jnp.float32),
        mesh=_sc_mesh,
        scratch_types=[
            pltpu.VMEM((CHUNK,), jnp.int32),
            pltpu.VMEM((CHUNK,), jnp.float32),
            pltpu.VMEM((CHUNK, 128), jnp.float32),
            pltpu.VMEM_SHARED((ACC_ROWS, 128), jnp.float32),
            pltpu.SemaphoreType.DMA,
        ],
    )


EH_DEG = 32 * CHUNK * 79
EH_DROW = 32 * CHUNK * 98
_hist_deg = _make_hist_sc(79)
_hist_drow = _make_hist_sc(98)


def _hist(idx, val, epad, hist_fn, zeros128):
    n = idx.shape[0]
    idxp = jnp.concatenate([idx, jnp.full((epad - n,), N, jnp.int32)])
    valp = jnp.concatenate([val, jnp.zeros((epad - n,), jnp.float32)])
    out = hist_fn(idxp, valp, zeros128)
    return out[0, :N, 0] + out[1, :N, 0]


def _norm_adj_edges(edge_index, zeros16):
    src = edge_index[0].astype(jnp.int32)
    dst = edge_index[1].astype(jnp.int32)
    self_loop = src == dst
    arow = jnp.where(self_loop, N, src)
    acol = dst
    arow = jnp.concatenate([arow, jnp.full((EA_PAD - E,), N, jnp.int32)])
    acol = jnp.concatenate([acol, jnp.zeros((EA_PAD - E,), jnp.int32)])
    notloop = (~self_loop).astype(jnp.float32)
    deg = _hist(dst, notloop, EH_DEG, _hist_deg, zeros16) + 1.0
    deg = jnp.maximum(deg, 1.0)
    d_isqrt = deg ** -0.5
    return arow, acol, d_isqrt


def _knn_graph(features, zeros16):
    xn = _row_normalize(features)
    xn_pad = jnp.pad(xn, ((0, NPAD - N), (0, 0)))
    topv_p, topi_p = _topk_sim(xn_pad)
    topv = topv_p[:N, :K]
    topi = jnp.minimum(topi_p[:N, :K], N - 1)
    rows = jnp.repeat(jnp.arange(N, dtype=jnp.int32), K)
    cols = topi.reshape(-1)
    sim = jnp.maximum(topv.reshape(-1), 0.0)
    r = jnp.concatenate([rows, cols])
    c = jnp.concatenate([cols, rows])
    v = jnp.concatenate([sim, sim])
    Drow = _hist(r, v, EH_DROW, _hist_drow, zeros16)
    Dis = jnp.maximum(Drow, 1e-12) ** -0.5
    srow = jnp.concatenate([r, jnp.full((ES_PAD - 2 * N * K,), N, jnp.int32)])
    scol = jnp.concatenate([c, jnp.zeros((ES_PAD - 2 * N * K,), jnp.int32)])
    sval = jnp.concatenate([v, jnp.zeros((ES_PAD - 2 * N * K,), jnp.float32)])
    return srow, scol, sval, Dis


_ZEROS_SLAB = None


def _propagate(H, arow, acol, d_isqrt, srow, scol, sval, Dis, zeros_slab):
    gA = d_isqrt[:, None] * H
    gS = Dis[:, None] * H
    outA, outS = _spmm_sc(gA, gS, arow, acol, srow, scol, sval, zeros_slab)
    AH = d_isqrt[:, None] * (outA[:N] + gA)
    SfH = Dis[:, None] * outS[:N]
    return AH, SfH


def kernel(features, edge_index, W0, b0, W1, b1, ws0, bs0, ws1, bs1, wk0, bk0, wk1, bk1):
    zeros_slab = jnp.zeros((ROWS_PER_SUB, 128), jnp.float32)
    arow, acol, d_isqrt = _norm_adj_edges(edge_index, zeros_slab)
    srow, scol, sval, Dis = _knn_graph(features, zeros_slab)
    H = features
    s = jax.nn.sigmoid(H @ ws0 + bs0)
    Kv = H @ wk0 + bk0
    AH, SfH = _propagate(H, arow, acol, d_isqrt, srow, scol, sval, Dis,
                         zeros_slab)
    PH = s[:, None] * AH + (1.0 - s)[:, None] * SfH + GAMMA * (Kv[:, None] * H)
    H = _pallas_matmul(PH, W0) + b0
    H = jax.nn.relu(H)
    s = jax.nn.sigmoid(H @ ws1 + bs1)
    Kv = H @ wk1 + bk1
    AH, SfH = _propagate(H, arow, acol, d_isqrt, srow, scol, sval, Dis,
                         zeros_slab)
    PH = s[:, None] * AH + (1.0 - s)[:, None] * SfH + GAMMA * (Kv[:, None] * H)
    H = _pallas_matmul(PH, W1) + b1
    return H
